# Initial kernel scaffold; baseline (speedup 1.0000x reference)
import functools

import jax
import jax.numpy as jnp
from jax import lax
from jax.experimental import pallas as pl
from jax.experimental.pallas import tpu as pltpu
from jax.experimental.pallas import tpu_sc as plsc

N = 10000
E = 320000
NT = 16
ROWS_PT = N // NT
ZCH = 125
C = 128

_f32 = jnp.float32


def _sc_mesh():
    return plsc.VectorSubcoreMesh(core_axis_name="c", subcore_axis_name="s")


def _deg_kernel(dst_hbm, out_hbm, idx_v, idx_t, ones_v, acc, sem):
    c = lax.axis_index("c")
    s = lax.axis_index("s")
    ept = E // (2 * NT)
    nfull = ept // C
    tail = ept - nfull * C

    def _z(i, _):
        ones_v[i, :] = jnp.zeros((16,), _f32)
        return 0
    lax.fori_loop(0, ZCH, _z, 0)
    for k in range(5):
        pltpu.sync_copy(ones_v.at[pl.ds(0, ZCH)],
                        acc.at[pl.ds(s * ROWS_PT + k * ZCH, ZCH)])

    def _o(i, _):
        ones_v[i, :] = jnp.full((16,), 1.0, _f32)
        return 0
    lax.fori_loop(0, C, _o, 0)
    plsc.subcore_barrier()

    ebase = (c * NT + s) * ept

    def _chunk(i, _):
        pltpu.sync_copy(dst_hbm.at[pl.ds(ebase + i * C, C)], idx_v)
        pltpu.sync_copy(ones_v.at[pl.ds(0, C)], acc.at[idx_v], add=True)
        return 0
    lax.fori_loop(0, nfull, _chunk, 0)
    if tail:
        pltpu.sync_copy(dst_hbm.at[pl.ds(ebase + nfull * C, tail)], idx_t)
        pltpu.sync_copy(ones_v.at[pl.ds(0, tail)], acc.at[idx_t], add=True)

    plsc.subcore_barrier()
    pltpu.sync_copy(acc.at[pl.ds(s * ROWS_PT, ROWS_PT)],
                    out_hbm.at[c, pl.ds(s * ROWS_PT, ROWS_PT)])


def _deg_partials(dst):
    return pl.kernel(
        _deg_kernel,
        out_type=jax.ShapeDtypeStruct((2, N, 16), _f32),
        mesh=_sc_mesh(),
        scratch_types=[
            pltpu.VMEM((C,), jnp.int32),
            pltpu.VMEM((16,), jnp.int32),
            pltpu.VMEM((ZCH, 16), _f32),
            pltpu.VMEM_SHARED((N, 16), _f32),
            pltpu.SemaphoreType.DMA,
        ],
    )(dst)


def _spmm_kernel(fc, hs_hbm, src_hbm, dst_hbm, out_hbm,
                 idx_s, idx_d, idx_st, idx_dt, rows, rows_t, zbuf, acc, sem):
    c = lax.axis_index("c")
    s = lax.axis_index("s")
    ept = E // NT
    nfull = ept // C
    tail = ept - nfull * C

    def _z(i, _):
        for j in range(fc // 16):
            zbuf[i, pl.ds(j * 16, 16)] = jnp.zeros((16,), _f32)
        return 0
    lax.fori_loop(0, ZCH, _z, 0)
    for k in range(5):
        pltpu.sync_copy(zbuf, acc.at[pl.ds(s * ROWS_PT + k * ZCH, ZCH)])
    plsc.subcore_barrier()

    ebase = s * ept

    def _chunk(i, _):
        off = ebase + i * C
        pltpu.sync_copy(src_hbm.at[pl.ds(off, C)], idx_s)
        pltpu.sync_copy(dst_hbm.at[pl.ds(off, C)], idx_d)
        pltpu.async_copy(hs_hbm.at[c, idx_s], rows, sem).wait()
        pltpu.sync_copy(rows, acc.at[idx_d], add=True)
        return 0
    lax.fori_loop(0, nfull, _chunk, 0)
    if tail:
        off = ebase + nfull * C
        pltpu.sync_copy(src_hbm.at[pl.ds(off, tail)], idx_st)
        pltpu.sync_copy(dst_hbm.at[pl.ds(off, tail)], idx_dt)
        pltpu.async_copy(hs_hbm.at[c, idx_st], rows_t, sem).wait()
        pltpu.sync_copy(rows_t, acc.at[idx_dt], add=True)

    plsc.subcore_barrier()
    pltpu.sync_copy(acc.at[pl.ds(s * ROWS_PT, ROWS_PT)],
                    out_hbm.at[c, pl.ds(s * ROWS_PT, ROWS_PT)])


def _spmm(hs, src, dst, fc):
    ept = E // NT
    tail = ept - (ept // C) * C
    return pl.kernel(
        functools.partial(_spmm_kernel, fc),
        out_type=jax.ShapeDtypeStruct((2, N, fc), _f32),
        mesh=_sc_mesh(),
        scratch_types=[
            pltpu.VMEM((C,), jnp.int32),
            pltpu.VMEM((C,), jnp.int32),
            pltpu.VMEM((tail,), jnp.int32),
            pltpu.VMEM((tail,), jnp.int32),
            pltpu.VMEM((C, fc), _f32),
            pltpu.VMEM((tail, fc), _f32),
            pltpu.VMEM((ZCH, fc), _f32),
            pltpu.VMEM_SHARED((N, fc), _f32),
            pltpu.SemaphoreType.DMA,
        ],
    )(hs, src, dst)


R = 1000
BN_S = 1.0 / jnp.sqrt(jnp.float32(1.0 + 1e-5))


def _dinv(degt_ref):
    d = degt_ref[0, :, 0:1] + degt_ref[1, :, 0:1] + 1.0
    return lax.rsqrt(d)


def _b1_body(x_ref, w1_ref, degt_ref, hs1_ref):
    dinv = _dinv(degt_ref)
    h = jnp.dot(x_ref[...], w1_ref[...], preferred_element_type=_f32)
    hs = dinv * h
    hs1_ref[0] = hs[:, :128]
    hs1_ref[1] = hs[:, 128:]


def _b2_body(agg_ref, hs_ref, degt_ref, b1_ref, g1_ref, be1_ref,
             w2_ref, wr_ref, br_ref, hs2_ref, r_ref):
    dinv = _dinv(degt_ref)
    s = jnp.concatenate([agg_ref[0] + hs_ref[0], agg_ref[1] + hs_ref[1]],
                        axis=1)
    v = dinv * s + b1_ref[...]
    x1 = jnp.maximum(g1_ref[...] * (v * BN_S) + be1_ref[...], 0.0)
    h2 = jnp.dot(x1, w2_ref[...], preferred_element_type=_f32)
    hs2 = dinv * h2
    hs2_ref[0] = hs2[:, :64]
    hs2_ref[1] = hs2[:, 64:]
    r_ref[...] = jnp.dot(x1, wr_ref[...], preferred_element_type=_f32) \
        + br_ref[...]


def _b3_body(agg_ref, hs_ref, degt_ref, b2_ref, g2_ref, be2_ref,
             r_ref, w3_ref, hs3_ref, x2_ref):
    dinv = _dinv(degt_ref)
    s = jnp.concatenate([agg_ref[0] + hs_ref[0], agg_ref[1] + hs_ref[1]],
                        axis=1)
    v = dinv * s + b2_ref[...]
    v = g2_ref[...] * (v * BN_S) + be2_ref[...]
    x2 = jnp.maximum(v + r_ref[...], 0.0)
    h3 = jnp.dot(x2, w3_ref[...], preferred_element_type=_f32)
    hs3 = dinv * h3
    hs3_ref[0] = hs3[:, :64]
    hs3_ref[1] = hs3[:, 64:]
    x2_ref[...] = x2


def _b4_body(agg_ref, hs_ref, degt_ref, b3_ref, g3_ref, be3_ref,
             x2_ref, out_ref):
    dinv = _dinv(degt_ref)
    s = jnp.concatenate([agg_ref[0] + hs_ref[0], agg_ref[1] + hs_ref[1]],
                        axis=1)
    v = dinv * s + b3_ref[...]
    v = g3_ref[...] * (v * BN_S) + be3_ref[...]
    out_ref[...] = jnp.maximum(v + x2_ref[...], 0.0)


def _rows(shape):
    return pl.BlockSpec((R,) + shape[1:], lambda i: (i,) + (0,) * (len(shape) - 1))


def _split_rows(fc):
    return pl.BlockSpec((2, R, fc), lambda i: (0, i, 0))


def _full(shape):
    return pl.BlockSpec(shape, lambda i: (0,) * len(shape))


def _tc_call(body, in_specs, out_specs, out_shapes):
    return pl.pallas_call(
        body,
        grid=(N // R,),
        in_specs=in_specs,
        out_specs=out_specs,
        out_shape=out_shapes,
    )


def kernel(x, edge_index, W1, b1, g1, be1, W2, b2, g2, be2, Wr, br,
           W3, b3, g3, be3):
    src = edge_index[0]
    dst = edge_index[1]
    b1r, g1r, be1r = b1.reshape(1, -1), g1.reshape(1, -1), be1.reshape(1, -1)
    b2r, g2r, be2r = b2.reshape(1, -1), g2.reshape(1, -1), be2.reshape(1, -1)
    b3r, g3r, be3r = b3.reshape(1, -1), g3.reshape(1, -1), be3.reshape(1, -1)
    brr = br.reshape(1, -1)

    degt = _deg_partials(dst)

    hs1 = _tc_call(
        _b1_body,
        [_rows((N, 128)), _full((128, 256)), _split_rows(16)],
        _split_rows(128),
        jax.ShapeDtypeStruct((2, N, 128), _f32),
    )(x, W1, degt)

    agg1 = _spmm(hs1, src, dst, 128)

    hs2, r = _tc_call(
        _b2_body,
        [_split_rows(128), _split_rows(128), _split_rows(16),
         _full((1, 256)), _full((1, 256)), _full((1, 256)),
         _full((256, 128)), _full((256, 128)), _full((1, 128))],
        [_split_rows(64), _rows((N, 128))],
        [jax.ShapeDtypeStruct((2, N, 64), _f32),
         jax.ShapeDtypeStruct((N, 128), _f32)],
    )(agg1, hs1, degt, b1r, g1r, be1r, W2, Wr, brr)

    agg2 = _spmm(hs2, src, dst, 64)

    hs3, x2 = _tc_call(
        _b3_body,
        [_split_rows(64), _split_rows(64), _split_rows(16),
         _full((1, 128)), _full((1, 128)), _full((1, 128)),
         _rows((N, 128)), _full((128, 128))],
        [_split_rows(64), _rows((N, 128))],
        [jax.ShapeDtypeStruct((2, N, 64), _f32),
         jax.ShapeDtypeStruct((N, 128), _f32)],
    )(agg2, hs2, degt, b2r, g2r, be2r, r, W3)

    agg3 = _spmm(hs3, src, dst, 64)

    x3 = _tc_call(
        _b4_body,
        [_split_rows(64), _split_rows(64), _split_rows(16),
         _full((1, 128)), _full((1, 128)), _full((1, 128)),
         _rows((N, 128))],
        _rows((N, 128)),
        jax.ShapeDtypeStruct((N, 128), _f32),
    )(agg3, hs3, degt, b3r, g3r, be3r, x2)

    return x3


# speedup vs baseline: 12.0880x; 12.0880x over previous
import functools

import jax
import jax.numpy as jnp
from jax import lax
from jax.experimental import pallas as pl
from jax.experimental.pallas import tpu as pltpu
from jax.experimental.pallas import tpu_sc as plsc

N = 10000
E = 320000
NT = 16
ROWS_PT = N // NT
ZCH = 125
WB = 632
C = 128
F = 128

_f32 = jnp.float32


def _sc_mesh():
    return plsc.VectorSubcoreMesh(core_axis_name="c", subcore_axis_name="s")


def _writeback(acc, out_hbm, s):
    base = pl.multiple_of(s * WB, 8)

    @pl.when(s < NT - 1)
    def _():
        pltpu.sync_copy(acc.at[pl.ds(base, WB)], out_hbm.at[pl.ds(base, WB)])

    @pl.when(s == NT - 1)
    def _():
        last = (NT - 1) * WB
        pltpu.sync_copy(acc.at[pl.ds(last, N - last)],
                        out_hbm.at[pl.ds(last, N - last)])


def _deg_kernel(dst_hbm, outa_hbm, outb_hbm, idx_v, idx_t, ones_v, acc, sem):
    c = lax.axis_index("c")
    s = lax.axis_index("s")
    ept = E // (2 * NT)
    nfull = ept // C
    tail = ept - nfull * C

    def _z(i, _):
        ones_v[i, :] = jnp.zeros((16,), _f32)
        return 0
    lax.fori_loop(0, ZCH, _z, 0)
    for k in range(5):
        pltpu.sync_copy(ones_v.at[pl.ds(0, ZCH)],
                        acc.at[pl.ds(s * ROWS_PT + k * ZCH, ZCH)])

    def _o(i, _):
        ones_v[i, :] = jnp.full((16,), 1.0, _f32)
        return 0
    lax.fori_loop(0, C, _o, 0)
    plsc.subcore_barrier()

    ebase = (c * NT + s) * ept

    def _chunk(i, _):
        pltpu.sync_copy(dst_hbm.at[pl.ds(ebase + i * C, C)], idx_v)
        pltpu.sync_copy(ones_v.at[pl.ds(0, C)], acc.at[idx_v], add=True)
        return 0
    lax.fori_loop(0, nfull, _chunk, 0)
    if tail:
        pltpu.sync_copy(dst_hbm.at[pl.ds(ebase + nfull * C, tail)], idx_t)
        pltpu.sync_copy(ones_v.at[pl.ds(0, tail)], acc.at[idx_t], add=True)

    plsc.subcore_barrier()

    @pl.when(c == 0)
    def _():
        _writeback(acc, outa_hbm, s)

    @pl.when(c == 1)
    def _():
        _writeback(acc, outb_hbm, s)


def _deg_partials(dst):
    return pl.kernel(
        _deg_kernel,
        out_type=[jax.ShapeDtypeStruct((N, 16), _f32),
                  jax.ShapeDtypeStruct((N, 16), _f32)],
        mesh=_sc_mesh(),
        scratch_types=[
            pltpu.VMEM((C,), jnp.int32),
            pltpu.VMEM((16,), jnp.int32),
            pltpu.VMEM((ZCH, 16), _f32),
            pltpu.VMEM_SHARED((N, 16), _f32),
            pltpu.SemaphoreType.DMA,
        ],
    )(dst)


def _spmm_kernel(edge_split, hsa_hbm, hsb_hbm, src_hbm, dst_hbm,
                 outa_hbm, outb_hbm,
                 idx_s, idx_d, idx_st, idx_dt, rows, rows_t, zbuf, acc, sem):
    c = lax.axis_index("c")
    s = lax.axis_index("s")
    ept = E // (2 * NT) if edge_split else E // NT
    nfull = ept // C
    tail = ept - nfull * C

    def _z(i, _):
        for j in range(F // 16):
            zbuf[i, pl.ds(j * 16, 16)] = jnp.zeros((16,), _f32)
        return 0
    lax.fori_loop(0, ZCH, _z, 0)
    for k in range(5):
        pltpu.sync_copy(zbuf, acc.at[pl.ds(s * ROWS_PT + k * ZCH, ZCH)])
    plsc.subcore_barrier()

    ebase = (c * NT + s) * ept if edge_split else s * ept

    def _gather(idx, buf):
        @pl.when(c == 0)
        def _():
            pltpu.async_copy(hsa_hbm.at[idx], buf, sem).wait()

        @pl.when(c == 1)
        def _():
            pltpu.async_copy(hsb_hbm.at[idx], buf, sem).wait()

    def _chunk(i, _):
        off = ebase + i * C
        pltpu.sync_copy(src_hbm.at[pl.ds(off, C)], idx_s)
        pltpu.sync_copy(dst_hbm.at[pl.ds(off, C)], idx_d)
        _gather(idx_s, rows)
        pltpu.sync_copy(rows, acc.at[idx_d], add=True)
        return 0
    lax.fori_loop(0, nfull, _chunk, 0)
    if tail:
        off = ebase + nfull * C
        pltpu.sync_copy(src_hbm.at[pl.ds(off, tail)], idx_st)
        pltpu.sync_copy(dst_hbm.at[pl.ds(off, tail)], idx_dt)
        _gather(idx_st, rows_t)
        pltpu.sync_copy(rows_t, acc.at[idx_dt], add=True)

    plsc.subcore_barrier()

    @pl.when(c == 0)
    def _():
        _writeback(acc, outa_hbm, s)

    @pl.when(c == 1)
    def _():
        _writeback(acc, outb_hbm, s)


def _spmm(hsa, hsb, src, dst, edge_split):
    ept = E // (2 * NT) if edge_split else E // NT
    tail = ept - (ept // C) * C
    return pl.kernel(
        functools.partial(_spmm_kernel, edge_split),
        out_type=[jax.ShapeDtypeStruct((N, F), _f32),
                  jax.ShapeDtypeStruct((N, F), _f32)],
        mesh=_sc_mesh(),
        scratch_types=[
            pltpu.VMEM((C,), jnp.int32),
            pltpu.VMEM((C,), jnp.int32),
            pltpu.VMEM((tail,), jnp.int32),
            pltpu.VMEM((tail,), jnp.int32),
            pltpu.VMEM((C, F), _f32),
            pltpu.VMEM((tail, F), _f32),
            pltpu.VMEM((ZCH, F), _f32),
            pltpu.VMEM_SHARED((N, F), _f32),
            pltpu.SemaphoreType.DMA,
        ],
    )(hsa, hsb, src, dst)


R = 1000
BN_S = float((1.0 + 1e-5) ** -0.5)


def _dinv(dega_ref, degb_ref):
    d = dega_ref[:, 0:1] + degb_ref[:, 0:1] + 1.0
    return lax.rsqrt(d)


def _b1_body(x_ref, w1_ref, dega_ref, degb_ref, hsa_ref, hsb_ref):
    dinv = _dinv(dega_ref, degb_ref)
    h = jnp.dot(x_ref[...], w1_ref[...], preferred_element_type=_f32)
    hs = dinv * h
    hsa_ref[...] = hs[:, :128]
    hsb_ref[...] = hs[:, 128:]


def _b2_body(agga_ref, aggb_ref, hsa_ref, hsb_ref, dega_ref, degb_ref,
             b1_ref, g1_ref, be1_ref, w2_ref, wr_ref, br_ref,
             hs2_ref, r_ref):
    dinv = _dinv(dega_ref, degb_ref)
    s = jnp.concatenate([agga_ref[...] + hsa_ref[...],
                         aggb_ref[...] + hsb_ref[...]], axis=1)
    v = dinv * s + b1_ref[...]
    x1 = jnp.maximum(g1_ref[...] * (v * BN_S) + be1_ref[...], 0.0)
    h2 = jnp.dot(x1, w2_ref[...], preferred_element_type=_f32)
    hs2_ref[...] = dinv * h2
    r_ref[...] = jnp.dot(x1, wr_ref[...], preferred_element_type=_f32) \
        + br_ref[...]


def _b3_body(agga_ref, aggb_ref, hs_ref, dega_ref, degb_ref,
             b2_ref, g2_ref, be2_ref, r_ref, w3_ref,
             hs3_ref, x2_ref):
    dinv = _dinv(dega_ref, degb_ref)
    s = agga_ref[...] + aggb_ref[...] + hs_ref[...]
    v = dinv * s + b2_ref[...]
    v = g2_ref[...] * (v * BN_S) + be2_ref[...]
    x2 = jnp.maximum(v + r_ref[...], 0.0)
    h3 = jnp.dot(x2, w3_ref[...], preferred_element_type=_f32)
    hs3_ref[...] = dinv * h3
    x2_ref[...] = x2


def _b4_body(agga_ref, aggb_ref, hs_ref, dega_ref, degb_ref,
             b3_ref, g3_ref, be3_ref, x2_ref, out_ref):
    dinv = _dinv(dega_ref, degb_ref)
    s = agga_ref[...] + aggb_ref[...] + hs_ref[...]
    v = dinv * s + b3_ref[...]
    v = g3_ref[...] * (v * BN_S) + be3_ref[...]
    out_ref[...] = jnp.maximum(v + x2_ref[...], 0.0)


def _rows(cols):
    return pl.BlockSpec((R, cols), lambda i: (i, 0))


def _full(shape):
    return pl.BlockSpec(shape, lambda i: (0,) * len(shape))


def _tc_call(body, in_specs, out_specs, out_shapes):
    return pl.pallas_call(
        body,
        grid=(N // R,),
        in_specs=in_specs,
        out_specs=out_specs,
        out_shape=out_shapes,
    )


def kernel(x, edge_index, W1, b1, g1, be1, W2, b2, g2, be2, Wr, br,
           W3, b3, g3, be3):
    src = edge_index[0]
    dst = edge_index[1]
    b1r, g1r, be1r = b1.reshape(1, -1), g1.reshape(1, -1), be1.reshape(1, -1)
    b2r, g2r, be2r = b2.reshape(1, -1), g2.reshape(1, -1), be2.reshape(1, -1)
    b3r, g3r, be3r = b3.reshape(1, -1), g3.reshape(1, -1), be3.reshape(1, -1)
    brr = br.reshape(1, -1)

    dega, degb = _deg_partials(dst)

    hs1a, hs1b = _tc_call(
        _b1_body,
        [_rows(128), _full((128, 256)), _rows(16), _rows(16)],
        [_rows(128), _rows(128)],
        [jax.ShapeDtypeStruct((N, 128), _f32),
         jax.ShapeDtypeStruct((N, 128), _f32)],
    )(x, W1, dega, degb)

    agg1a, agg1b = _spmm(hs1a, hs1b, src, dst, edge_split=False)

    hs2, r = _tc_call(
        _b2_body,
        [_rows(128), _rows(128), _rows(128), _rows(128), _rows(16), _rows(16),
         _full((1, 256)), _full((1, 256)), _full((1, 256)),
         _full((256, 128)), _full((256, 128)), _full((1, 128))],
        [_rows(128), _rows(128)],
        [jax.ShapeDtypeStruct((N, 128), _f32),
         jax.ShapeDtypeStruct((N, 128), _f32)],
    )(agg1a, agg1b, hs1a, hs1b, dega, degb, b1r, g1r, be1r, W2, Wr, brr)

    agg2a, agg2b = _spmm(hs2, hs2, src, dst, edge_split=True)

    hs3, x2 = _tc_call(
        _b3_body,
        [_rows(128), _rows(128), _rows(128), _rows(16), _rows(16),
         _full((1, 128)), _full((1, 128)), _full((1, 128)),
         _rows(128), _full((128, 128))],
        [_rows(128), _rows(128)],
        [jax.ShapeDtypeStruct((N, 128), _f32),
         jax.ShapeDtypeStruct((N, 128), _f32)],
    )(agg2a, agg2b, hs2, dega, degb, b2r, g2r, be2r, r, W3)

    agg3a, agg3b = _spmm(hs3, hs3, src, dst, edge_split=True)

    x3 = _tc_call(
        _b4_body,
        [_rows(128), _rows(128), _rows(128), _rows(16), _rows(16),
         _full((1, 128)), _full((1, 128)), _full((1, 128)),
         _rows(128)],
        _rows(128),
        jax.ShapeDtypeStruct((N, 128), _f32),
    )(agg3a, agg3b, hs3, dega, degb, b3r, g3r, be3r, x2)

    return x3


# speedup vs baseline: 18.5034x; 1.5307x over previous
import functools

import jax
import jax.numpy as jnp
from jax import lax
from jax.experimental import pallas as pl
from jax.experimental.pallas import tpu as pltpu
from jax.experimental.pallas import tpu_sc as plsc

N = 10000
E = 320000
NT = 16
ROWS_PT = N // NT
ZCH = 125
WB = 632
C = 128
F = 128

_f32 = jnp.float32


def _sc_mesh():
    return plsc.VectorSubcoreMesh(core_axis_name="c", subcore_axis_name="s")


def _writeback(acc, out_hbm, s):
    base = pl.multiple_of(s * WB, 8)

    @pl.when(s < NT - 1)
    def _():
        pltpu.sync_copy(acc.at[pl.ds(base, WB)], out_hbm.at[pl.ds(base, WB)])

    @pl.when(s == NT - 1)
    def _():
        last = (NT - 1) * WB
        pltpu.sync_copy(acc.at[pl.ds(last, N - last)],
                        out_hbm.at[pl.ds(last, N - last)])


def _deg_kernel(dst_hbm, outa_hbm, outb_hbm, idx_v, idx_t, ones_v, acc, sem):
    c = lax.axis_index("c")
    s = lax.axis_index("s")
    ept = E // (2 * NT)
    nfull = ept // C
    tail = ept - nfull * C

    def _z(i, _):
        ones_v[i, :] = jnp.zeros((16,), _f32)
        return 0
    lax.fori_loop(0, ZCH, _z, 0)
    for k in range(5):
        pltpu.sync_copy(ones_v.at[pl.ds(0, ZCH)],
                        acc.at[pl.ds(s * ROWS_PT + k * ZCH, ZCH)])

    def _o(i, _):
        ones_v[i, :] = jnp.full((16,), 1.0, _f32)
        return 0
    lax.fori_loop(0, C, _o, 0)
    plsc.subcore_barrier()

    ebase = (c * NT + s) * ept

    def _chunk(i, _):
        pltpu.sync_copy(dst_hbm.at[pl.ds(ebase + i * C, C)], idx_v)
        pltpu.sync_copy(ones_v.at[pl.ds(0, C)], acc.at[idx_v], add=True)
        return 0
    lax.fori_loop(0, nfull, _chunk, 0)
    if tail:
        pltpu.sync_copy(dst_hbm.at[pl.ds(ebase + nfull * C, tail)], idx_t)
        pltpu.sync_copy(ones_v.at[pl.ds(0, tail)], acc.at[idx_t], add=True)

    plsc.subcore_barrier()

    @pl.when(c == 0)
    def _():
        _writeback(acc, outa_hbm, s)

    @pl.when(c == 1)
    def _():
        _writeback(acc, outb_hbm, s)


def _deg_partials(dst):
    return pl.kernel(
        _deg_kernel,
        out_type=[jax.ShapeDtypeStruct((N, 16), _f32),
                  jax.ShapeDtypeStruct((N, 16), _f32)],
        mesh=_sc_mesh(),
        scratch_types=[
            pltpu.VMEM((C,), jnp.int32),
            pltpu.VMEM((16,), jnp.int32),
            pltpu.VMEM((ZCH, 16), _f32),
            pltpu.VMEM_SHARED((N, 16), _f32),
            pltpu.SemaphoreType.DMA,
        ],
    )(dst)


NB = 3


def _spmm_kernel(edge_split, hsa_hbm, hsb_hbm, src_hbm, dst_hbm,
                 outa_hbm, outb_hbm,
                 is0, is1, is2, id0, id1, id2, idx_st, idx_dt,
                 rows0, rows1, rows2, acc,
                 gs0, gs1, gs2, ss0, ss1, ss2, tsem):
    c = lax.axis_index("c")
    s = lax.axis_index("s")
    ept = E // (2 * NT) if edge_split else E // NT
    nfull = ept // C
    tail = ept - nfull * C
    assert nfull % NB == 0
    idx_s = [is0, is1, is2]
    idx_d = [id0, id1, id2]
    rows = [rows0, rows1, rows2]
    gsem = [gs0, gs1, gs2]
    ssem = [ss0, ss1, ss2]

    def _z(i, _):
        for j in range(F // 16):
            rows0[i, pl.ds(j * 16, 16)] = jnp.zeros((16,), _f32)
        return 0
    lax.fori_loop(0, C, _z, 0)
    for k in range(4):
        pltpu.sync_copy(rows0, acc.at[pl.ds(s * ROWS_PT + k * C, C)])
    pltpu.sync_copy(rows0.at[pl.ds(0, ROWS_PT - 4 * C)],
                    acc.at[pl.ds(s * ROWS_PT + 4 * C, ROWS_PT - 4 * C)])
    plsc.subcore_barrier()

    ebase = (c * NT + s) * ept if edge_split else s * ept

    def _load_idx(off, b):
        pltpu.sync_copy(src_hbm.at[pl.ds(off, C)], idx_s[b])
        pltpu.sync_copy(dst_hbm.at[pl.ds(off, C)], idx_d[b])

    def _gstart(b):
        @pl.when(c == 0)
        def _():
            pltpu.async_copy(hsa_hbm.at[idx_s[b]], rows[b], gsem[b])

        @pl.when(c == 1)
        def _():
            pltpu.async_copy(hsb_hbm.at[idx_s[b]], rows[b], gsem[b])

    def _gwait(b):
        pltpu.make_async_copy(hsa_hbm.at[idx_s[b]], rows[b], gsem[b]).wait()

    def _sstart(b):
        pltpu.async_copy(rows[b], acc.at[idx_d[b]], ssem[b], add=True)

    def _swait(b):
        pltpu.make_async_copy(rows[b], acc.at[idx_d[b]], ssem[b]).wait()

    _load_idx(ebase, 0)
    _gstart(0)
    _load_idx(ebase + C, 1)
    _gstart(1)
    _load_idx(ebase + 2 * C, 2)
    _gstart(2)
    _gwait(0)
    _sstart(0)

    def _group(k, _):
        for b in range(NB):
            i0 = k * NB + b
            _swait(b)
            _load_idx(ebase + i0 * C, b)
            _gstart(b)
            bb = (b + 1) % NB
            _gwait(bb)
            _sstart(bb)
        return 0
    lax.fori_loop(1, nfull // NB, _group, 0)

    _gwait(1)
    _sstart(1)
    _gwait(2)
    _sstart(2)
    _swait(0)
    _swait(1)
    _swait(2)

    if tail:
        off = ebase + nfull * C
        pltpu.sync_copy(src_hbm.at[pl.ds(off, tail)], idx_st)
        pltpu.sync_copy(dst_hbm.at[pl.ds(off, tail)], idx_dt)

        rows_t = rows0.at[pl.ds(0, tail)]

        @pl.when(c == 0)
        def _():
            pltpu.async_copy(hsa_hbm.at[idx_st], rows_t, tsem).wait()

        @pl.when(c == 1)
        def _():
            pltpu.async_copy(hsb_hbm.at[idx_st], rows_t, tsem).wait()

        pltpu.sync_copy(rows_t, acc.at[idx_dt], add=True)

    plsc.subcore_barrier()

    @pl.when(c == 0)
    def _():
        _writeback(acc, outa_hbm, s)

    @pl.when(c == 1)
    def _():
        _writeback(acc, outb_hbm, s)


def _spmm(hsa, hsb, src, dst, edge_split):
    ept = E // (2 * NT) if edge_split else E // NT
    tail = ept - (ept // C) * C
    return pl.kernel(
        functools.partial(_spmm_kernel, edge_split),
        out_type=[jax.ShapeDtypeStruct((N, F), _f32),
                  jax.ShapeDtypeStruct((N, F), _f32)],
        mesh=_sc_mesh(),
        scratch_types=[
            pltpu.VMEM((C,), jnp.int32),
            pltpu.VMEM((C,), jnp.int32),
            pltpu.VMEM((C,), jnp.int32),
            pltpu.VMEM((C,), jnp.int32),
            pltpu.VMEM((C,), jnp.int32),
            pltpu.VMEM((C,), jnp.int32),
            pltpu.VMEM((tail,), jnp.int32),
            pltpu.VMEM((tail,), jnp.int32),
            pltpu.VMEM((C, F), _f32),
            pltpu.VMEM((C, F), _f32),
            pltpu.VMEM((C, F), _f32),
            pltpu.VMEM_SHARED((N, F), _f32),
            pltpu.SemaphoreType.DMA,
            pltpu.SemaphoreType.DMA,
            pltpu.SemaphoreType.DMA,
            pltpu.SemaphoreType.DMA,
            pltpu.SemaphoreType.DMA,
            pltpu.SemaphoreType.DMA,
            pltpu.SemaphoreType.DMA,
        ],
    )(hsa, hsb, src, dst)


R = 1000
BN_S = float((1.0 + 1e-5) ** -0.5)


def _dinv(dega_ref, degb_ref):
    d = dega_ref[:, 0:1] + degb_ref[:, 0:1] + 1.0
    return lax.rsqrt(d)


def _b1_body(x_ref, w1_ref, dega_ref, degb_ref, hsa_ref, hsb_ref):
    dinv = _dinv(dega_ref, degb_ref)
    h = jnp.dot(x_ref[...], w1_ref[...], preferred_element_type=_f32)
    hs = dinv * h
    hsa_ref[...] = hs[:, :128]
    hsb_ref[...] = hs[:, 128:]


def _b2_body(agga_ref, aggb_ref, hsa_ref, hsb_ref, dega_ref, degb_ref,
             b1_ref, g1_ref, be1_ref, w2_ref, wr_ref, br_ref,
             hs2_ref, r_ref):
    dinv = _dinv(dega_ref, degb_ref)
    s = jnp.concatenate([agga_ref[...] + hsa_ref[...],
                         aggb_ref[...] + hsb_ref[...]], axis=1)
    v = dinv * s + b1_ref[...]
    x1 = jnp.maximum(g1_ref[...] * (v * BN_S) + be1_ref[...], 0.0)
    h2 = jnp.dot(x1, w2_ref[...], preferred_element_type=_f32)
    hs2_ref[...] = dinv * h2
    r_ref[...] = jnp.dot(x1, wr_ref[...], preferred_element_type=_f32) \
        + br_ref[...]


def _b3_body(agga_ref, aggb_ref, hs_ref, dega_ref, degb_ref,
             b2_ref, g2_ref, be2_ref, r_ref, w3_ref,
             hs3_ref, x2_ref):
    dinv = _dinv(dega_ref, degb_ref)
    s = agga_ref[...] + aggb_ref[...] + hs_ref[...]
    v = dinv * s + b2_ref[...]
    v = g2_ref[...] * (v * BN_S) + be2_ref[...]
    x2 = jnp.maximum(v + r_ref[...], 0.0)
    h3 = jnp.dot(x2, w3_ref[...], preferred_element_type=_f32)
    hs3_ref[...] = dinv * h3
    x2_ref[...] = x2


def _b4_body(agga_ref, aggb_ref, hs_ref, dega_ref, degb_ref,
             b3_ref, g3_ref, be3_ref, x2_ref, out_ref):
    dinv = _dinv(dega_ref, degb_ref)
    s = agga_ref[...] + aggb_ref[...] + hs_ref[...]
    v = dinv * s + b3_ref[...]
    v = g3_ref[...] * (v * BN_S) + be3_ref[...]
    out_ref[...] = jnp.maximum(v + x2_ref[...], 0.0)


def _rows(cols):
    return pl.BlockSpec((R, cols), lambda i: (i, 0))


def _full(shape):
    return pl.BlockSpec(shape, lambda i: (0,) * len(shape))


def _tc_call(body, in_specs, out_specs, out_shapes):
    return pl.pallas_call(
        body,
        grid=(N // R,),
        in_specs=in_specs,
        out_specs=out_specs,
        out_shape=out_shapes,
    )


def kernel(x, edge_index, W1, b1, g1, be1, W2, b2, g2, be2, Wr, br,
           W3, b3, g3, be3):
    src = edge_index[0]
    dst = edge_index[1]
    b1r, g1r, be1r = b1.reshape(1, -1), g1.reshape(1, -1), be1.reshape(1, -1)
    b2r, g2r, be2r = b2.reshape(1, -1), g2.reshape(1, -1), be2.reshape(1, -1)
    b3r, g3r, be3r = b3.reshape(1, -1), g3.reshape(1, -1), be3.reshape(1, -1)
    brr = br.reshape(1, -1)

    dega, degb = _deg_partials(dst)

    hs1a, hs1b = _tc_call(
        _b1_body,
        [_rows(128), _full((128, 256)), _rows(16), _rows(16)],
        [_rows(128), _rows(128)],
        [jax.ShapeDtypeStruct((N, 128), _f32),
         jax.ShapeDtypeStruct((N, 128), _f32)],
    )(x, W1, dega, degb)

    agg1a, agg1b = _spmm(hs1a, hs1b, src, dst, edge_split=False)

    hs2, r = _tc_call(
        _b2_body,
        [_rows(128), _rows(128), _rows(128), _rows(128), _rows(16), _rows(16),
         _full((1, 256)), _full((1, 256)), _full((1, 256)),
         _full((256, 128)), _full((256, 128)), _full((1, 128))],
        [_rows(128), _rows(128)],
        [jax.ShapeDtypeStruct((N, 128), _f32),
         jax.ShapeDtypeStruct((N, 128), _f32)],
    )(agg1a, agg1b, hs1a, hs1b, dega, degb, b1r, g1r, be1r, W2, Wr, brr)

    agg2a, agg2b = _spmm(hs2, hs2, src, dst, edge_split=True)

    hs3, x2 = _tc_call(
        _b3_body,
        [_rows(128), _rows(128), _rows(128), _rows(16), _rows(16),
         _full((1, 128)), _full((1, 128)), _full((1, 128)),
         _rows(128), _full((128, 128))],
        [_rows(128), _rows(128)],
        [jax.ShapeDtypeStruct((N, 128), _f32),
         jax.ShapeDtypeStruct((N, 128), _f32)],
    )(agg2a, agg2b, hs2, dega, degb, b2r, g2r, be2r, r, W3)

    agg3a, agg3b = _spmm(hs3, hs3, src, dst, edge_split=True)

    x3 = _tc_call(
        _b4_body,
        [_rows(128), _rows(128), _rows(128), _rows(16), _rows(16),
         _full((1, 128)), _full((1, 128)), _full((1, 128)),
         _rows(128)],
        _rows(128),
        jax.ShapeDtypeStruct((N, 128), _f32),
    )(agg3a, agg3b, hs3, dega, degb, b3r, g3r, be3r, x2)

    return x3


# speedup vs baseline: 28.0744x; 1.5173x over previous
import functools

import jax
import jax.numpy as jnp
from jax import lax
from jax.experimental import pallas as pl
from jax.experimental.pallas import tpu as pltpu
from jax.experimental.pallas import tpu_sc as plsc

N = 10000
E = 320000
NT = 16
ROWS_PT = N // NT
ZCH = 125
WB = 632
C = 128
F = 128

_f32 = jnp.float32


def _sc_mesh():
    return plsc.VectorSubcoreMesh(core_axis_name="c", subcore_axis_name="s")


def _writeback(acc, out_hbm, s):
    base = pl.multiple_of(s * WB, 8)

    @pl.when(s < NT - 1)
    def _():
        pltpu.sync_copy(acc.at[pl.ds(base, WB)], out_hbm.at[pl.ds(base, WB)])

    @pl.when(s == NT - 1)
    def _():
        last = (NT - 1) * WB
        pltpu.sync_copy(acc.at[pl.ds(last, N - last)],
                        out_hbm.at[pl.ds(last, N - last)])


def _deg_kernel(dst_hbm, outa_hbm, outb_hbm, qi0, qi1, qi2, idx_t, ones_v,
                acc, qs0, qs1, qs2, ss0, ss1, ss2, tsem):
    c = lax.axis_index("c")
    s = lax.axis_index("s")
    ept = E // (2 * NT)
    nfull = ept // C
    tail = ept - nfull * C
    assert nfull % 3 == 0
    idx = [qi0, qi1, qi2]
    isem = [qs0, qs1, qs2]
    ssem = [ss0, ss1, ss2]

    def _z(i, _):
        ones_v[i, :] = jnp.zeros((16,), _f32)
        return 0
    lax.fori_loop(0, C, _z, 0)
    for k in range(5):
        pltpu.sync_copy(ones_v.at[pl.ds(0, ZCH)],
                        acc.at[pl.ds(s * ROWS_PT + k * ZCH, ZCH)])

    def _o(i, _):
        ones_v[i, :] = jnp.full((16,), 1.0, _f32)
        return 0
    lax.fori_loop(0, C, _o, 0)
    plsc.subcore_barrier()

    ebase = (c * NT + s) * ept
    emax = ebase + (nfull - 1) * C

    def _istart(off, b):
        off = jnp.minimum(off, emax)
        pltpu.async_copy(dst_hbm.at[pl.ds(off, C)], idx[b], isem[b])

    def _iwait(b):
        pltpu.make_async_copy(dst_hbm.at[pl.ds(0, C)], idx[b],
                              isem[b]).wait()

    def _sstart(b):
        pltpu.async_copy(ones_v, acc.at[idx[b]], ssem[b], add=True)

    def _swait(b):
        pltpu.make_async_copy(ones_v, acc.at[idx[b]], ssem[b]).wait()

    _istart(ebase, 0)
    _istart(ebase + C, 1)
    _iwait(0)
    _sstart(0)
    _istart(ebase + 2 * C, 2)
    _iwait(1)
    _sstart(1)
    _swait(0)
    _istart(ebase + 3 * C, 0)
    _iwait(2)
    _sstart(2)

    def _group(k, _):
        for b in range(3):
            i0 = 3 * k + b
            bb = (b + 1) % 3
            _swait(bb)
            _istart(ebase + (i0 + 1) * C, bb)
            _iwait(b)
            _sstart(b)
        return 0
    lax.fori_loop(1, nfull // 3, _group, 0)

    _swait(1)
    _swait(2)
    _iwait(0)

    if tail:
        pltpu.sync_copy(dst_hbm.at[pl.ds(ebase + nfull * C, tail)], idx_t)
        pltpu.sync_copy(ones_v.at[pl.ds(0, tail)], acc.at[idx_t], add=True)

    plsc.subcore_barrier()

    @pl.when(c == 0)
    def _():
        _writeback(acc, outa_hbm, s)

    @pl.when(c == 1)
    def _():
        _writeback(acc, outb_hbm, s)


def _deg_partials(dst):
    return pl.kernel(
        _deg_kernel,
        out_type=[jax.ShapeDtypeStruct((N, 16), _f32),
                  jax.ShapeDtypeStruct((N, 16), _f32)],
        mesh=_sc_mesh(),
        scratch_types=[
            pltpu.VMEM((C,), jnp.int32),
            pltpu.VMEM((C,), jnp.int32),
            pltpu.VMEM((C,), jnp.int32),
            pltpu.VMEM((16,), jnp.int32),
            pltpu.VMEM((C, 16), _f32),
            pltpu.VMEM_SHARED((N, 16), _f32),
            pltpu.SemaphoreType.DMA,
            pltpu.SemaphoreType.DMA,
            pltpu.SemaphoreType.DMA,
            pltpu.SemaphoreType.DMA,
            pltpu.SemaphoreType.DMA,
            pltpu.SemaphoreType.DMA,
            pltpu.SemaphoreType.DMA,
        ],
    )(dst)


NB = 3


def _spmm_kernel(edge_split, hsa_hbm, hsb_hbm, src_hbm, dst_hbm,
                 outa_hbm, outb_hbm,
                 is0, is1, is2, id0, id1, id2, idx_st, idx_dt,
                 rows0, rows1, rows2, acc,
                 gs0, gs1, gs2, ss0, ss1, ss2, qs0, qs1, qs2, tsem):
    c = lax.axis_index("c")
    s = lax.axis_index("s")
    ept = E // (2 * NT) if edge_split else E // NT
    nfull = ept // C
    tail = ept - nfull * C
    assert nfull % NB == 0
    idx_s = [is0, is1, is2]
    idx_d = [id0, id1, id2]
    rows = [rows0, rows1, rows2]
    gsem = [gs0, gs1, gs2]
    ssem = [ss0, ss1, ss2]
    isem = [qs0, qs1, qs2]

    def _z(i, _):
        for j in range(F // 16):
            rows0[i, pl.ds(j * 16, 16)] = jnp.zeros((16,), _f32)
        return 0
    lax.fori_loop(0, C, _z, 0)
    for k in range(4):
        pltpu.sync_copy(rows0, acc.at[pl.ds(s * ROWS_PT + k * C, C)])
    pltpu.sync_copy(rows0.at[pl.ds(0, ROWS_PT - 4 * C)],
                    acc.at[pl.ds(s * ROWS_PT + 4 * C, ROWS_PT - 4 * C)])
    plsc.subcore_barrier()

    ebase = (c * NT + s) * ept if edge_split else s * ept

    emax = ebase + (nfull - 1) * C

    def _istart(off, b):
        off = jnp.minimum(off, emax)
        pltpu.async_copy(src_hbm.at[pl.ds(off, C)], idx_s[b], isem[b])
        pltpu.async_copy(dst_hbm.at[pl.ds(off, C)], idx_d[b], isem[b])

    def _iwait(b):
        pltpu.make_async_copy(src_hbm.at[pl.ds(0, C)], idx_s[b],
                              isem[b]).wait()
        pltpu.make_async_copy(dst_hbm.at[pl.ds(0, C)], idx_d[b],
                              isem[b]).wait()

    def _gstart(b):
        @pl.when(c == 0)
        def _():
            pltpu.async_copy(hsa_hbm.at[idx_s[b]], rows[b], gsem[b])

        @pl.when(c == 1)
        def _():
            pltpu.async_copy(hsb_hbm.at[idx_s[b]], rows[b], gsem[b])

    def _gwait(b):
        pltpu.make_async_copy(hsa_hbm.at[idx_s[b]], rows[b], gsem[b]).wait()

    def _sstart(b):
        pltpu.async_copy(rows[b], acc.at[idx_d[b]], ssem[b], add=True)

    def _swait(b):
        pltpu.make_async_copy(rows[b], acc.at[idx_d[b]], ssem[b]).wait()

    _istart(ebase, 0)
    _istart(ebase + C, 1)
    _istart(ebase + 2 * C, 2)
    _iwait(0)
    _gstart(0)
    _iwait(1)
    _gstart(1)
    _iwait(2)
    _gstart(2)
    _gwait(0)
    _sstart(0)
    _istart(ebase + 3 * C, 0)

    def _group(k, _):
        for b in range(NB):
            i0 = k * NB + b
            _swait(b)
            _iwait(b)
            _gstart(b)
            bb = (b + 1) % NB
            _gwait(bb)
            _sstart(bb)
            _istart(ebase + (i0 + 1) * C, bb)
        return 0
    lax.fori_loop(1, nfull // NB, _group, 0)

    _gwait(1)
    _sstart(1)
    _gwait(2)
    _sstart(2)
    _swait(0)
    _swait(1)
    _swait(2)
    _iwait(0)

    if tail:
        off = ebase + nfull * C
        pltpu.sync_copy(src_hbm.at[pl.ds(off, tail)], idx_st)
        pltpu.sync_copy(dst_hbm.at[pl.ds(off, tail)], idx_dt)

        rows_t = rows0.at[pl.ds(0, tail)]

        @pl.when(c == 0)
        def _():
            pltpu.async_copy(hsa_hbm.at[idx_st], rows_t, tsem).wait()

        @pl.when(c == 1)
        def _():
            pltpu.async_copy(hsb_hbm.at[idx_st], rows_t, tsem).wait()

        pltpu.sync_copy(rows_t, acc.at[idx_dt], add=True)

    plsc.subcore_barrier()

    @pl.when(c == 0)
    def _():
        _writeback(acc, outa_hbm, s)

    @pl.when(c == 1)
    def _():
        _writeback(acc, outb_hbm, s)


def _spmm(hsa, hsb, src, dst, edge_split):
    ept = E // (2 * NT) if edge_split else E // NT
    tail = ept - (ept // C) * C
    return pl.kernel(
        functools.partial(_spmm_kernel, edge_split),
        out_type=[jax.ShapeDtypeStruct((N, F), _f32),
                  jax.ShapeDtypeStruct((N, F), _f32)],
        mesh=_sc_mesh(),
        scratch_types=[
            pltpu.VMEM((C,), jnp.int32),
            pltpu.VMEM((C,), jnp.int32),
            pltpu.VMEM((C,), jnp.int32),
            pltpu.VMEM((C,), jnp.int32),
            pltpu.VMEM((C,), jnp.int32),
            pltpu.VMEM((C,), jnp.int32),
            pltpu.VMEM((tail,), jnp.int32),
            pltpu.VMEM((tail,), jnp.int32),
            pltpu.VMEM((C, F), _f32),
            pltpu.VMEM((C, F), _f32),
            pltpu.VMEM((C, F), _f32),
            pltpu.VMEM_SHARED((N, F), _f32),
            pltpu.SemaphoreType.DMA,
            pltpu.SemaphoreType.DMA,
            pltpu.SemaphoreType.DMA,
            pltpu.SemaphoreType.DMA,
            pltpu.SemaphoreType.DMA,
            pltpu.SemaphoreType.DMA,
            pltpu.SemaphoreType.DMA,
            pltpu.SemaphoreType.DMA,
            pltpu.SemaphoreType.DMA,
            pltpu.SemaphoreType.DMA,
        ],
    )(hsa, hsb, src, dst)


R = 1000
BN_S = float((1.0 + 1e-5) ** -0.5)


def _dinv(dega_ref, degb_ref):
    d = dega_ref[:, 0:1] + degb_ref[:, 0:1] + 1.0
    return lax.rsqrt(d)


def _b1_body(x_ref, w1_ref, dega_ref, degb_ref, hsa_ref, hsb_ref):
    dinv = _dinv(dega_ref, degb_ref)
    h = jnp.dot(x_ref[...], w1_ref[...], preferred_element_type=_f32)
    hs = dinv * h
    hsa_ref[...] = hs[:, :128]
    hsb_ref[...] = hs[:, 128:]


def _b2_body(agga_ref, aggb_ref, hsa_ref, hsb_ref, dega_ref, degb_ref,
             b1_ref, g1_ref, be1_ref, w2_ref, wr_ref, br_ref,
             hs2_ref, r_ref):
    dinv = _dinv(dega_ref, degb_ref)
    s = jnp.concatenate([agga_ref[...] + hsa_ref[...],
                         aggb_ref[...] + hsb_ref[...]], axis=1)
    v = dinv * s + b1_ref[...]
    x1 = jnp.maximum(g1_ref[...] * (v * BN_S) + be1_ref[...], 0.0)
    h2 = jnp.dot(x1, w2_ref[...], preferred_element_type=_f32)
    hs2_ref[...] = dinv * h2
    r_ref[...] = jnp.dot(x1, wr_ref[...], preferred_element_type=_f32) \
        + br_ref[...]


def _b3_body(agga_ref, aggb_ref, hs_ref, dega_ref, degb_ref,
             b2_ref, g2_ref, be2_ref, r_ref, w3_ref,
             hs3_ref, x2_ref):
    dinv = _dinv(dega_ref, degb_ref)
    s = agga_ref[...] + aggb_ref[...] + hs_ref[...]
    v = dinv * s + b2_ref[...]
    v = g2_ref[...] * (v * BN_S) + be2_ref[...]
    x2 = jnp.maximum(v + r_ref[...], 0.0)
    h3 = jnp.dot(x2, w3_ref[...], preferred_element_type=_f32)
    hs3_ref[...] = dinv * h3
    x2_ref[...] = x2


def _b4_body(agga_ref, aggb_ref, hs_ref, dega_ref, degb_ref,
             b3_ref, g3_ref, be3_ref, x2_ref, out_ref):
    dinv = _dinv(dega_ref, degb_ref)
    s = agga_ref[...] + aggb_ref[...] + hs_ref[...]
    v = dinv * s + b3_ref[...]
    v = g3_ref[...] * (v * BN_S) + be3_ref[...]
    out_ref[...] = jnp.maximum(v + x2_ref[...], 0.0)


def _rows(cols):
    return pl.BlockSpec((R, cols), lambda i: (i, 0))


def _full(shape):
    return pl.BlockSpec(shape, lambda i: (0,) * len(shape))


def _tc_call(body, in_specs, out_specs, out_shapes):
    return pl.pallas_call(
        body,
        grid=(N // R,),
        in_specs=in_specs,
        out_specs=out_specs,
        out_shape=out_shapes,
    )


def kernel(x, edge_index, W1, b1, g1, be1, W2, b2, g2, be2, Wr, br,
           W3, b3, g3, be3):
    src = edge_index[0]
    dst = edge_index[1]
    b1r, g1r, be1r = b1.reshape(1, -1), g1.reshape(1, -1), be1.reshape(1, -1)
    b2r, g2r, be2r = b2.reshape(1, -1), g2.reshape(1, -1), be2.reshape(1, -1)
    b3r, g3r, be3r = b3.reshape(1, -1), g3.reshape(1, -1), be3.reshape(1, -1)
    brr = br.reshape(1, -1)

    dega, degb = _deg_partials(dst)

    hs1a, hs1b = _tc_call(
        _b1_body,
        [_rows(128), _full((128, 256)), _rows(16), _rows(16)],
        [_rows(128), _rows(128)],
        [jax.ShapeDtypeStruct((N, 128), _f32),
         jax.ShapeDtypeStruct((N, 128), _f32)],
    )(x, W1, dega, degb)

    agg1a, agg1b = _spmm(hs1a, hs1b, src, dst, edge_split=False)

    hs2, r = _tc_call(
        _b2_body,
        [_rows(128), _rows(128), _rows(128), _rows(128), _rows(16), _rows(16),
         _full((1, 256)), _full((1, 256)), _full((1, 256)),
         _full((256, 128)), _full((256, 128)), _full((1, 128))],
        [_rows(128), _rows(128)],
        [jax.ShapeDtypeStruct((N, 128), _f32),
         jax.ShapeDtypeStruct((N, 128), _f32)],
    )(agg1a, agg1b, hs1a, hs1b, dega, degb, b1r, g1r, be1r, W2, Wr, brr)

    agg2a, agg2b = _spmm(hs2, hs2, src, dst, edge_split=True)

    hs3, x2 = _tc_call(
        _b3_body,
        [_rows(128), _rows(128), _rows(128), _rows(16), _rows(16),
         _full((1, 128)), _full((1, 128)), _full((1, 128)),
         _rows(128), _full((128, 128))],
        [_rows(128), _rows(128)],
        [jax.ShapeDtypeStruct((N, 128), _f32),
         jax.ShapeDtypeStruct((N, 128), _f32)],
    )(agg2a, agg2b, hs2, dega, degb, b2r, g2r, be2r, r, W3)

    agg3a, agg3b = _spmm(hs3, hs3, src, dst, edge_split=True)

    x3 = _tc_call(
        _b4_body,
        [_rows(128), _rows(128), _rows(128), _rows(16), _rows(16),
         _full((1, 128)), _full((1, 128)), _full((1, 128)),
         _rows(128)],
        _rows(128),
        jax.ShapeDtypeStruct((N, 128), _f32),
    )(agg3a, agg3b, hs3, dega, degb, b3r, g3r, be3r, x2)

    return x3


# speedup vs baseline: 33.5400x; 1.1947x over previous
import functools

import jax
import jax.numpy as jnp
from jax import lax
from jax.experimental import pallas as pl
from jax.experimental.pallas import tpu as pltpu
from jax.experimental.pallas import tpu_sc as plsc

N = 10000
E = 320000
NT = 16
ROWS_PT = N // NT
ZCH = 125
WB = 632
C = 128
F = 128

_f32 = jnp.float32


def _sc_mesh():
    return plsc.VectorSubcoreMesh(core_axis_name="c", subcore_axis_name="s")


def _writeback(acc, out_hbm, s):
    base = pl.multiple_of(s * WB, 8)

    @pl.when(s < NT - 1)
    def _():
        pltpu.sync_copy(acc.at[pl.ds(base, WB)], out_hbm.at[pl.ds(base, WB)])

    @pl.when(s == NT - 1)
    def _():
        last = (NT - 1) * WB
        pltpu.sync_copy(acc.at[pl.ds(last, N - last)],
                        out_hbm.at[pl.ds(last, N - last)])


def _deg_kernel(dst_hbm, outa_hbm, outb_hbm, qi0, qi1, qi2, idx_t, ones_v,
                acc, qs0, qs1, qs2, ss0, ss1, ss2, tsem):
    c = lax.axis_index("c")
    s = lax.axis_index("s")
    ept = E // (2 * NT)
    nfull = ept // C
    tail = ept - nfull * C
    assert nfull % 3 == 0
    idx = [qi0, qi1, qi2]
    isem = [qs0, qs1, qs2]
    ssem = [ss0, ss1, ss2]

    def _z(i, _):
        ones_v[i, :] = jnp.zeros((16,), _f32)
        return 0
    lax.fori_loop(0, C, _z, 0)
    for k in range(5):
        pltpu.sync_copy(ones_v.at[pl.ds(0, ZCH)],
                        acc.at[pl.ds(s * ROWS_PT + k * ZCH, ZCH)])

    def _o(i, _):
        ones_v[i, :] = jnp.full((16,), 1.0, _f32)
        return 0
    lax.fori_loop(0, C, _o, 0)
    plsc.subcore_barrier()

    ebase = (c * NT + s) * ept
    emax = ebase + (nfull - 1) * C

    def _istart(off, b):
        off = jnp.minimum(off, emax)
        pltpu.async_copy(dst_hbm.at[pl.ds(off, C)], idx[b], isem[b])

    def _iwait(b):
        pltpu.make_async_copy(dst_hbm.at[pl.ds(0, C)], idx[b],
                              isem[b]).wait()

    def _sstart(b):
        pltpu.async_copy(ones_v, acc.at[idx[b]], ssem[b], add=True)

    def _swait(b):
        pltpu.make_async_copy(ones_v, acc.at[idx[b]], ssem[b]).wait()

    _istart(ebase, 0)
    _istart(ebase + C, 1)
    _iwait(0)
    _sstart(0)
    _istart(ebase + 2 * C, 2)
    _iwait(1)
    _sstart(1)
    _swait(0)
    _istart(ebase + 3 * C, 0)
    _iwait(2)
    _sstart(2)

    def _group(k, _):
        for b in range(3):
            i0 = 3 * k + b
            bb = (b + 1) % 3
            _swait(bb)
            _istart(ebase + (i0 + 1) * C, bb)
            _iwait(b)
            _sstart(b)
        return 0
    lax.fori_loop(1, nfull // 3, _group, 0)

    _swait(1)
    _swait(2)
    _iwait(0)

    if tail:
        pltpu.sync_copy(dst_hbm.at[pl.ds(ebase + nfull * C, tail)], idx_t)
        pltpu.sync_copy(ones_v.at[pl.ds(0, tail)], acc.at[idx_t], add=True)

    plsc.subcore_barrier()

    @pl.when(c == 0)
    def _():
        _writeback(acc, outa_hbm, s)

    @pl.when(c == 1)
    def _():
        _writeback(acc, outb_hbm, s)


def _deg_partials(dst):
    return pl.kernel(
        _deg_kernel,
        out_type=[jax.ShapeDtypeStruct((N, 16), _f32),
                  jax.ShapeDtypeStruct((N, 16), _f32)],
        mesh=_sc_mesh(),
        scratch_types=[
            pltpu.VMEM((C,), jnp.int32),
            pltpu.VMEM((C,), jnp.int32),
            pltpu.VMEM((C,), jnp.int32),
            pltpu.VMEM((16,), jnp.int32),
            pltpu.VMEM((C, 16), _f32),
            pltpu.VMEM_SHARED((N, 16), _f32),
            pltpu.SemaphoreType.DMA,
            pltpu.SemaphoreType.DMA,
            pltpu.SemaphoreType.DMA,
            pltpu.SemaphoreType.DMA,
            pltpu.SemaphoreType.DMA,
            pltpu.SemaphoreType.DMA,
            pltpu.SemaphoreType.DMA,
        ],
    )(dst)


NB = 3


def _spmm_kernel(edge_split, hsa_hbm, hsb_hbm, src_hbm, dst_hbm,
                 outa_hbm, outb_hbm,
                 is0, is1, is2, id0, id1, id2, idx_st, idx_dt,
                 rows0, rows1, rows2, acc,
                 gs0, gs1, gs2, ss0, ss1, ss2, qs0, qs1, qs2, tsem):
    c = lax.axis_index("c")
    s = lax.axis_index("s")
    ept = E // (2 * NT) if edge_split else E // NT
    nfull = ept // C
    tail = ept - nfull * C
    assert nfull % NB == 0
    idx_s = [is0, is1, is2]
    idx_d = [id0, id1, id2]
    rows = [rows0, rows1, rows2]
    gsem = [gs0, gs1, gs2]
    ssem = [ss0, ss1, ss2]
    isem = [qs0, qs1, qs2]

    def _z(i, _):
        for j in range(F // 16):
            rows0[i, pl.ds(j * 16, 16)] = jnp.zeros((16,), _f32)
        return 0
    lax.fori_loop(0, C, _z, 0)
    for k in range(4):
        pltpu.sync_copy(rows0, acc.at[pl.ds(s * ROWS_PT + k * C, C)])
    pltpu.sync_copy(rows0.at[pl.ds(0, ROWS_PT - 4 * C)],
                    acc.at[pl.ds(s * ROWS_PT + 4 * C, ROWS_PT - 4 * C)])
    plsc.subcore_barrier()

    ebase = (c * NT + s) * ept if edge_split else s * ept

    emax = ebase + (nfull - 1) * C

    def _istart(off, b):
        off = jnp.minimum(off, emax)
        pltpu.async_copy(src_hbm.at[pl.ds(off, C)], idx_s[b], isem[b])
        pltpu.async_copy(dst_hbm.at[pl.ds(off, C)], idx_d[b], isem[b])

    def _iwait(b):
        pltpu.make_async_copy(src_hbm.at[pl.ds(0, C)], idx_s[b],
                              isem[b]).wait()
        pltpu.make_async_copy(dst_hbm.at[pl.ds(0, C)], idx_d[b],
                              isem[b]).wait()

    def _gstart(b):
        @pl.when(c == 0)
        def _():
            pltpu.async_copy(hsa_hbm.at[idx_s[b]], rows[b], gsem[b])

        @pl.when(c == 1)
        def _():
            pltpu.async_copy(hsb_hbm.at[idx_s[b]], rows[b], gsem[b])

    def _gwait(b):
        pltpu.make_async_copy(hsa_hbm.at[idx_s[b]], rows[b], gsem[b]).wait()

    def _sstart(b):
        pltpu.async_copy(rows[b], acc.at[idx_d[b]], ssem[b], add=True)

    def _swait(b):
        pltpu.make_async_copy(rows[b], acc.at[idx_d[b]], ssem[b]).wait()

    _istart(ebase, 0)
    _istart(ebase + C, 1)
    _istart(ebase + 2 * C, 2)
    _iwait(0)
    _gstart(0)
    _iwait(1)
    _gstart(1)
    _iwait(2)
    _gstart(2)
    _gwait(0)
    _sstart(0)
    _istart(ebase + 3 * C, 0)

    def _group(k, _):
        for b in range(NB):
            i0 = k * NB + b
            _swait(b)
            _iwait(b)
            _gstart(b)
            bb = (b + 1) % NB
            _gwait(bb)
            _sstart(bb)
            _istart(ebase + (i0 + 1) * C, bb)
        return 0
    lax.fori_loop(1, nfull // NB, _group, 0)

    _gwait(1)
    _sstart(1)
    _gwait(2)
    _sstart(2)
    _swait(0)
    _swait(1)
    _swait(2)
    _iwait(0)

    if tail:
        off = ebase + nfull * C
        pltpu.sync_copy(src_hbm.at[pl.ds(off, tail)], idx_st)
        pltpu.sync_copy(dst_hbm.at[pl.ds(off, tail)], idx_dt)

        rows_t = rows0.at[pl.ds(0, tail)]

        @pl.when(c == 0)
        def _():
            pltpu.async_copy(hsa_hbm.at[idx_st], rows_t, tsem).wait()

        @pl.when(c == 1)
        def _():
            pltpu.async_copy(hsb_hbm.at[idx_st], rows_t, tsem).wait()

        pltpu.sync_copy(rows_t, acc.at[idx_dt], add=True)

    plsc.subcore_barrier()

    @pl.when(c == 0)
    def _():
        _writeback(acc, outa_hbm, s)

    @pl.when(c == 1)
    def _():
        _writeback(acc, outb_hbm, s)


def _spmm(z, src, dst):
    ept = E // (2 * NT)
    tail = ept - (ept // C) * C
    return pl.kernel(
        functools.partial(_spmm_kernel, True),
        out_type=[jax.ShapeDtypeStruct((N, F), _f32),
                  jax.ShapeDtypeStruct((N, F), _f32)],
        mesh=_sc_mesh(),
        scratch_types=[
            pltpu.VMEM((C,), jnp.int32),
            pltpu.VMEM((C,), jnp.int32),
            pltpu.VMEM((C,), jnp.int32),
            pltpu.VMEM((C,), jnp.int32),
            pltpu.VMEM((C,), jnp.int32),
            pltpu.VMEM((C,), jnp.int32),
            pltpu.VMEM((tail,), jnp.int32),
            pltpu.VMEM((tail,), jnp.int32),
            pltpu.VMEM((C, F), _f32),
            pltpu.VMEM((C, F), _f32),
            pltpu.VMEM((C, F), _f32),
            pltpu.VMEM_SHARED((N, F), _f32),
            pltpu.SemaphoreType.DMA,
            pltpu.SemaphoreType.DMA,
            pltpu.SemaphoreType.DMA,
            pltpu.SemaphoreType.DMA,
            pltpu.SemaphoreType.DMA,
            pltpu.SemaphoreType.DMA,
            pltpu.SemaphoreType.DMA,
            pltpu.SemaphoreType.DMA,
            pltpu.SemaphoreType.DMA,
            pltpu.SemaphoreType.DMA,
        ],
    )(z, z, src, dst)


R = 1000
BN_S = float((1.0 + 1e-5) ** -0.5)


def _dinv(dega_ref, degb_ref):
    d = dega_ref[:, 0:1] + degb_ref[:, 0:1] + 1.0
    return lax.rsqrt(d)


def _k1_body(x_ref, dega_ref, degb_ref, z1_ref):
    z1_ref[...] = _dinv(dega_ref, degb_ref) * x_ref[...]


def _k2_body(pa_ref, pb_ref, z1_ref, dega_ref, degb_ref,
             w1_ref, b1_ref, g1_ref, be1_ref, w2_ref, wr_ref, br_ref,
             hs2_ref, r_ref):
    dinv = _dinv(dega_ref, degb_ref)
    q1 = dinv * (pa_ref[...] + pb_ref[...] + z1_ref[...])
    v = jnp.dot(q1, w1_ref[...], preferred_element_type=_f32) + b1_ref[...]
    x1 = jnp.maximum(g1_ref[...] * (v * BN_S) + be1_ref[...], 0.0)
    h2 = jnp.dot(x1, w2_ref[...], preferred_element_type=_f32)
    hs2_ref[...] = dinv * h2
    r_ref[...] = jnp.dot(x1, wr_ref[...], preferred_element_type=_f32) \
        + br_ref[...]


def _k3_body(pa_ref, pb_ref, hs2_ref, dega_ref, degb_ref,
             b2_ref, g2_ref, be2_ref, r_ref, z3_ref, x2_ref):
    dinv = _dinv(dega_ref, degb_ref)
    s2 = pa_ref[...] + pb_ref[...] + hs2_ref[...]
    v = dinv * s2 + b2_ref[...]
    v = g2_ref[...] * (v * BN_S) + be2_ref[...]
    x2 = jnp.maximum(v + r_ref[...], 0.0)
    z3_ref[...] = dinv * x2
    x2_ref[...] = x2


def _k4_body(pa_ref, pb_ref, z3_ref, dega_ref, degb_ref,
             w3_ref, b3_ref, g3_ref, be3_ref, x2_ref, out_ref):
    dinv = _dinv(dega_ref, degb_ref)
    q3 = dinv * (pa_ref[...] + pb_ref[...] + z3_ref[...])
    v = jnp.dot(q3, w3_ref[...], preferred_element_type=_f32) + b3_ref[...]
    v = g3_ref[...] * (v * BN_S) + be3_ref[...]
    out_ref[...] = jnp.maximum(v + x2_ref[...], 0.0)


def _rows(cols):
    return pl.BlockSpec((R, cols), lambda i: (i, 0))


def _full(shape):
    return pl.BlockSpec(shape, lambda i: (0,) * len(shape))


def _tc_call(body, in_specs, out_specs, out_shapes):
    return pl.pallas_call(
        body,
        grid=(N // R,),
        in_specs=in_specs,
        out_specs=out_specs,
        out_shape=out_shapes,
    )


def kernel(x, edge_index, W1, b1, g1, be1, W2, b2, g2, be2, Wr, br,
           W3, b3, g3, be3):
    src = edge_index[0]
    dst = edge_index[1]
    b1r, g1r, be1r = b1.reshape(1, -1), g1.reshape(1, -1), be1.reshape(1, -1)
    b2r, g2r, be2r = b2.reshape(1, -1), g2.reshape(1, -1), be2.reshape(1, -1)
    b3r, g3r, be3r = b3.reshape(1, -1), g3.reshape(1, -1), be3.reshape(1, -1)
    brr = br.reshape(1, -1)

    dega, degb = _deg_partials(dst)

    z1 = _tc_call(
        _k1_body,
        [_rows(128), _rows(16), _rows(16)],
        _rows(128),
        jax.ShapeDtypeStruct((N, 128), _f32),
    )(x, dega, degb)

    p1a, p1b = _spmm(z1, src, dst)

    hs2, r = _tc_call(
        _k2_body,
        [_rows(128), _rows(128), _rows(128), _rows(16), _rows(16),
         _full((128, 256)), _full((1, 256)), _full((1, 256)), _full((1, 256)),
         _full((256, 128)), _full((256, 128)), _full((1, 128))],
        [_rows(128), _rows(128)],
        [jax.ShapeDtypeStruct((N, 128), _f32),
         jax.ShapeDtypeStruct((N, 128), _f32)],
    )(p1a, p1b, z1, dega, degb, W1, b1r, g1r, be1r, W2, Wr, brr)

    p2a, p2b = _spmm(hs2, src, dst)

    z3, x2 = _tc_call(
        _k3_body,
        [_rows(128), _rows(128), _rows(128), _rows(16), _rows(16),
         _full((1, 128)), _full((1, 128)), _full((1, 128)), _rows(128)],
        [_rows(128), _rows(128)],
        [jax.ShapeDtypeStruct((N, 128), _f32),
         jax.ShapeDtypeStruct((N, 128), _f32)],
    )(p2a, p2b, hs2, dega, degb, b2r, g2r, be2r, r)

    p3a, p3b = _spmm(z3, src, dst)

    x3 = _tc_call(
        _k4_body,
        [_rows(128), _rows(128), _rows(128), _rows(16), _rows(16),
         _full((128, 128)), _full((1, 128)), _full((1, 128)), _full((1, 128)),
         _rows(128)],
        _rows(128),
        jax.ShapeDtypeStruct((N, 128), _f32),
    )(p3a, p3b, z3, dega, degb, W3, b3r, g3r, be3r, x2)

    return x3
